# baseline (device time: 27605 ns/iter reference)
import jax
import jax.numpy as jnp
from jax import lax
from jax.experimental import pallas as pl
from jax.experimental.pallas import tpu as pltpu

SCALE = 64 ** -0.5
NC = 8
HALF = 128
CH = HALF // NC


def _make_body(b_sz, s_loc, h, d):
    hd = h * d

    def body(q_ref, k_ref, v_ref, o_ref, kvl, kvr, mstat, dstat,
             sx, rx, sy, ry):
        my_x = lax.axis_index("x")
        my_y = lax.axis_index("y")
        xn = (1 - my_x, my_y)
        yn = (my_x, 1 - my_y)
        hy = my_y * HALF

        barrier = pltpu.get_barrier_semaphore()
        for nbr in (xn, yn):
            pl.semaphore_signal(
                barrier, inc=1, device_id=nbr,
                device_id_type=pl.DeviceIdType.MESH,
            )
        pl.semaphore_wait(barrier, 2)

        for b in range(b_sz):
            kvl[0, b] = k_ref[b].astype(jnp.bfloat16)
            kvl[1, b] = v_ref[b].astype(jnp.bfloat16)

        xch = []
        for c in range(NC):
            rows = pl.ds(hy + c * CH, CH)
            r = pltpu.make_async_remote_copy(
                src_ref=kvl.at[:, :, rows, :],
                dst_ref=kvr.at[:, :, rows, :],
                send_sem=sx.at[c],
                recv_sem=rx.at[c],
                device_id=xn,
                device_id_type=pl.DeviceIdType.MESH,
            )
            r.start()
            xch.append(r)

        fwd = []
        for c in range(NC):
            rows = pl.ds(hy + c * CH, CH)
            fwd.append(pltpu.make_async_remote_copy(
                src_ref=kvr.at[:, :, rows, :],
                dst_ref=kvr.at[:, :, rows, :],
                send_sem=sy.at[c],
                recv_sem=ry.at[c],
                device_id=yn,
                device_id_type=pl.DeviceIdType.MESH,
            ))

        def flash_unit(b, i, kv_buf, nrows, first):
            cols = slice(i * d, (i + 1) * d)
            q = q_ref[b, :, cols].astype(jnp.bfloat16)
            kh = kv_buf[0, b, :nrows, cols]
            vh = kv_buf[1, b, :nrows, cols]
            s = lax.dot_general(
                q, kh, (((1,), (1,)), ((), ())),
                preferred_element_type=jnp.float32,
            ) * SCALE
            mc = jnp.max(s, axis=1, keepdims=True)
            if first:
                p = jnp.exp(s - mc)
                dn = jnp.sum(p, axis=1, keepdims=True)
                o = lax.dot_general(
                    p.astype(jnp.bfloat16), vh, (((1,), (0,)), ((), ())),
                    preferred_element_type=jnp.float32,
                )
                o_ref[b, :, cols] = o
                mstat[b, :, i:i + 1] = mc
                dstat[b, :, i:i + 1] = dn
            else:
                mo = mstat[b, :, i:i + 1]
                do = dstat[b, :, i:i + 1]
                mn = jnp.maximum(mo, mc)
                p = jnp.exp(s - mn)
                corr = jnp.exp(mo - mn)
                dn = do * corr + jnp.sum(p, axis=1, keepdims=True)
                o = o_ref[b, :, cols] * corr + lax.dot_general(
                    p.astype(jnp.bfloat16), vh, (((1,), (0,)), ((), ())),
                    preferred_element_type=jnp.float32,
                )
                o_ref[b, :, cols] = o / dn

        units = [(b, i) for b in range(b_sz) for i in range(h)]
        per_fwd = max(1, len(units) // NC)
        fc = 0
        for u, (b, i) in enumerate(units):
            flash_unit(b, i, kvl, s_loc, True)
            if (u + 1) % per_fwd == 0 and fc < NC:
                xch[fc].wait_recv()
                fwd[fc].start()
                fc += 1
        while fc < NC:
            xch[fc].wait_recv()
            fwd[fc].start()
            fc += 1

        for c in range(NC):
            fwd[c].wait_recv()
        for b, i in units:
            flash_unit(b, i, kvr, s_loc, False)

        for c in range(NC):
            xch[c].wait_send()
            fwd[c].wait_send()

    return body


def kernel(Q, K, V):
    b, s, h, d = Q.shape
    hd = h * d
    Q3 = Q.reshape(b, s, hd)
    K3 = K.reshape(b, s, hd)
    V3 = V.reshape(b, s, hd)

    out3 = pl.pallas_call(
        _make_body(b, s, h, d),
        out_shape=jax.ShapeDtypeStruct((b, s, hd), jnp.float32),
        in_specs=[pl.BlockSpec(memory_space=pltpu.VMEM)] * 3,
        out_specs=pl.BlockSpec(memory_space=pltpu.VMEM),
        scratch_shapes=[
            pltpu.VMEM((2, b, s, hd), jnp.bfloat16),
            pltpu.VMEM((2, b, s, hd), jnp.bfloat16),
            pltpu.VMEM((b, s, h), jnp.float32),
            pltpu.VMEM((b, s, h), jnp.float32),
            pltpu.SemaphoreType.DMA((NC,)),
            pltpu.SemaphoreType.DMA((NC,)),
            pltpu.SemaphoreType.DMA((NC,)),
            pltpu.SemaphoreType.DMA((NC,)),
        ],
        compiler_params=pltpu.CompilerParams(collective_id=0),
    )(Q3, K3, V3)
    return out3.reshape(b, s, h, d)


# device time: 24130 ns/iter; 1.1440x vs baseline; 1.1440x over previous
import os

import jax
import jax.numpy as jnp
from jax import lax
from jax.experimental import pallas as pl
from jax.experimental.pallas import tpu as pltpu

KMODE = os.environ.get("KMODE", "full")

SCALE = 64 ** -0.5
NC = 8
HALF = 128
CH = HALF // NC
COMM = KMODE != "nocomm"


def _make_body(b_sz, s_loc, h, d):
    def body(q_ref, k_ref, v_ref, o_ref, kvl, kvr, sx, rx, sy, ry):
        my_x = lax.axis_index("x")
        my_y = lax.axis_index("y")
        xn = (1 - my_x, my_y)
        yn = (my_x, 1 - my_y)
        hy = my_y * HALF

        barrier = pltpu.get_barrier_semaphore()
        for nbr in (xn, yn):
            pl.semaphore_signal(
                barrier, inc=1, device_id=nbr,
                device_id_type=pl.DeviceIdType.MESH,
            )
        pl.semaphore_wait(barrier, 2)

        for b in range(b_sz):
            kvl[0, b] = k_ref[b].astype(jnp.bfloat16)
            kvl[1, b] = v_ref[b].astype(jnp.bfloat16)

        xch = []
        for c in range(NC):
            rows = pl.ds(hy + c * CH, CH)
            r = pltpu.make_async_remote_copy(
                src_ref=kvl.at[:, :, rows, :],
                dst_ref=kvr.at[:, :, rows, :],
                send_sem=sx.at[c],
                recv_sem=rx.at[c],
                device_id=xn,
                device_id_type=pl.DeviceIdType.MESH,
            )
            if COMM:
                r.start()
            xch.append(r)

        fwd = []
        for c in range(NC):
            rows = pl.ds(hy + c * CH, CH)
            fwd.append(pltpu.make_async_remote_copy(
                src_ref=kvr.at[:, :, rows, :],
                dst_ref=kvr.at[:, :, rows, :],
                send_sem=sy.at[c],
                recv_sem=ry.at[c],
                device_id=yn,
                device_id_type=pl.DeviceIdType.MESH,
            ))

        if COMM:
            for c in range(NC):
                xch[c].wait_recv()
                fwd[c].start()
            for c in range(NC):
                fwd[c].wait_recv()
        kvrem = kvr if COMM else kvl

        for b in range(b_sz):
            q = q_ref[b].astype(jnp.bfloat16)
            kl = kvl[0, b]
            vl = kvl[1, b]
            kr = kvrem[0, b]
            vr = kvrem[1, b]
            for i in range(h):
                cols = slice(i * d, (i + 1) * d)
                qh = q[:, cols]
                s1 = lax.dot_general(
                    qh, kl[:, cols], (((1,), (1,)), ((), ())),
                    preferred_element_type=jnp.float32,
                ) * SCALE
                s2 = lax.dot_general(
                    qh, kr[:, cols], (((1,), (1,)), ((), ())),
                    preferred_element_type=jnp.float32,
                ) * SCALE
                m = jnp.maximum(
                    jnp.max(s1, axis=1, keepdims=True),
                    jnp.max(s2, axis=1, keepdims=True),
                )
                p1 = jnp.exp(s1 - m)
                p2 = jnp.exp(s2 - m)
                den = (jnp.sum(p1, axis=1, keepdims=True)
                       + jnp.sum(p2, axis=1, keepdims=True))
                o = lax.dot_general(
                    p1.astype(jnp.bfloat16), vl[:, cols],
                    (((1,), (0,)), ((), ())),
                    preferred_element_type=jnp.float32,
                ) + lax.dot_general(
                    p2.astype(jnp.bfloat16), vr[:, cols],
                    (((1,), (0,)), ((), ())),
                    preferred_element_type=jnp.float32,
                )
                o_ref[b, :, cols] = o / den

        if COMM:
            for c in range(NC):
                xch[c].wait_send()
                fwd[c].wait_send()

    return body


def kernel(Q, K, V):
    b, s, h, d = Q.shape
    hd = h * d
    Q3 = Q.reshape(b, s, hd)
    K3 = K.reshape(b, s, hd)
    V3 = V.reshape(b, s, hd)

    out3 = pl.pallas_call(
        _make_body(b, s, h, d),
        out_shape=jax.ShapeDtypeStruct((b, s, hd), jnp.float32),
        in_specs=[pl.BlockSpec(memory_space=pltpu.VMEM)] * 3,
        out_specs=pl.BlockSpec(memory_space=pltpu.VMEM),
        scratch_shapes=[
            pltpu.VMEM((2, b, s, hd), jnp.bfloat16),
            pltpu.VMEM((2, b, s, hd), jnp.bfloat16),
            pltpu.SemaphoreType.DMA((NC,)),
            pltpu.SemaphoreType.DMA((NC,)),
            pltpu.SemaphoreType.DMA((NC,)),
            pltpu.SemaphoreType.DMA((NC,)),
        ],
        compiler_params=pltpu.CompilerParams(collective_id=0),
    )(Q3, K3, V3)
    return out3.reshape(b, s, h, d)
